# baseline (device time: 97443 ns/iter reference)
import functools

import jax
import jax.numpy as jnp
from jax import lax
from jax.experimental import pallas as pl
from jax.experimental.pallas import tpu as pltpu

N_DEV = 4
B, S, D = 2, 256, 512
H_PER = 4
DH = 64
EPS = 1e-5


def _dot(a, b, trans_b=False):
    dn = (((1,), (1 if trans_b else 0,)), ((), ()))
    return lax.dot_general(
        a.astype(jnp.bfloat16), b.astype(jnp.bfloat16), dn,
        preferred_element_type=jnp.float32,
    )


def _ln_mod(xb, scale_row, shift_row):
    m = jnp.mean(xb, axis=-1, keepdims=True)
    c = xb - m
    v = jnp.mean(c * c, axis=-1, keepdims=True)
    xn = c * lax.rsqrt(v + EPS)
    return xn * (1.0 + scale_row) + shift_row


def kernel(x, Wq, Wk, Wv, Wo, t_emb, W_mod, W_ff1, W_ff2):
    def body(x_ref, wq_ref, wk_ref, wv_ref, wo_ref, temb_ref, wmod_ref,
             wff1_ref, wff2_ref, out_ref, comm_ref, send_sems, recv_sems):
        my = lax.axis_index("i")
        left = lax.rem(my + N_DEV - 1, N_DEV)
        right = lax.rem(my + 1, N_DEV)

        barrier_sem = pltpu.get_barrier_semaphore()
        for nbr in (left, right):
            pl.semaphore_signal(
                barrier_sem, inc=1,
                device_id=(nbr,), device_id_type=pl.DeviceIdType.MESH,
            )
        pl.semaphore_wait(barrier_sem, 2)

        mod = lax.dot_general(
            temb_ref[:, :], wmod_ref[:, :], (((1,), (0,)), ((), ())),
            preferred_element_type=jnp.float32,
        )

        def mod_row(b, k):
            return mod[b:b + 1, k * D:(k + 1) * D]

        wq = wq_ref[:, :]
        wk = wk_ref[:, :]
        wv = wv_ref[:, :]
        wo = wo_ref[:, :]

        x0 = [x_ref[b] for b in range(B)]
        for b in range(B):
            xm = _ln_mod(x0[b], mod_row(b, 0), mod_row(b, 1))
            q = _dot(xm, wq)
            k = _dot(xm, wk)
            v = _dot(xm, wv)
            o_heads = []
            for h in range(H_PER):
                sl = slice(h * DH, (h + 1) * DH)
                s = _dot(q[:, sl], k[:, sl], trans_b=True) * 0.125
                s_max = jnp.max(s, axis=-1, keepdims=True)
                p = jnp.exp(s - s_max)
                p = p / jnp.sum(p, axis=-1, keepdims=True)
                o_heads.append(_dot(p, v[:, sl]))
            o = jnp.concatenate(o_heads, axis=1)
            comm_ref[0, b] = _dot(o, wo)

        acc1 = [comm_ref[0, b] for b in range(B)]
        for hop in range(N_DEV - 1):
            rdma = pltpu.make_async_remote_copy(
                src_ref=comm_ref.at[hop],
                dst_ref=comm_ref.at[hop + 1],
                send_sem=send_sems.at[hop],
                recv_sem=recv_sems.at[hop],
                device_id=(right,),
                device_id_type=pl.DeviceIdType.MESH,
            )
            rdma.start()
            rdma.wait()
            for b in range(B):
                acc1[b] = acc1[b] + comm_ref[hop + 1, b]

        x1 = [x0[b] + mod_row(b, 2) * acc1[b] for b in range(B)]
        wff1 = wff1_ref[:, :]
        wff2 = wff2_ref[:, :]
        for b in range(B):
            xm2 = _ln_mod(x1[b], mod_row(b, 3), mod_row(b, 4))
            h1 = _dot(xm2, wff1)
            h1 = h1 * jax.nn.sigmoid(h1)
            comm_ref[4, b] = _dot(h1, wff2)

        acc2 = [comm_ref[4, b] for b in range(B)]
        for hop in range(N_DEV - 1):
            rdma = pltpu.make_async_remote_copy(
                src_ref=comm_ref.at[4 + hop],
                dst_ref=comm_ref.at[4 + hop + 1],
                send_sem=send_sems.at[3 + hop],
                recv_sem=recv_sems.at[3 + hop],
                device_id=(right,),
                device_id_type=pl.DeviceIdType.MESH,
            )
            rdma.start()
            rdma.wait()
            for b in range(B):
                acc2[b] = acc2[b] + comm_ref[4 + hop + 1, b]

        for b in range(B):
            out_ref[b] = x1[b] + mod_row(b, 5) * acc2[b]

    return pl.pallas_call(
        body,
        out_shape=jax.ShapeDtypeStruct((B, S, D), jnp.float32),
        in_specs=[pl.BlockSpec(memory_space=pltpu.VMEM)] * 9,
        out_specs=pl.BlockSpec(memory_space=pltpu.VMEM),
        scratch_shapes=[
            pltpu.VMEM((8, B, S, D), jnp.float32),
            pltpu.SemaphoreType.DMA((6,)),
            pltpu.SemaphoreType.DMA((6,)),
        ],
        compiler_params=pltpu.CompilerParams(collective_id=0),
    )(x, Wq, Wk, Wv, Wo, t_emb, W_mod, W_ff1, W_ff2)


# device time: 19265 ns/iter; 5.0580x vs baseline; 5.0580x over previous
import jax
import jax.numpy as jnp
from jax import lax
from jax.experimental import pallas as pl
from jax.experimental.pallas import tpu as pltpu

N_DEV = 4
B, S, D = 2, 256, 512
H_PER = 4
DH = 64
EPS = 1e-5

A0, A1, B0, B1, C0, C1, D0, D1 = range(8)


def _dot(a, b, trans_b=False):
    dn = (((1,), (1 if trans_b else 0,)), ((), ()))
    return lax.dot_general(
        a.astype(jnp.bfloat16), b.astype(jnp.bfloat16), dn,
        preferred_element_type=jnp.float32,
    )


def _ln_mod(xb, scale_row, shift_row):
    m = jnp.mean(xb, axis=-1, keepdims=True)
    c = xb - m
    v = jnp.mean(c * c, axis=-1, keepdims=True)
    xn = c * lax.rsqrt(v + EPS)
    return xn * (1.0 + scale_row) + shift_row


def kernel(x, Wq, Wk, Wv, Wo, t_emb, W_mod, W_ff1, W_ff2):
    def body(x_ref, wq_ref, wk_ref, wv_ref, wo_ref, temb_ref, wmod_ref,
             wff1_ref, wff2_ref, out_ref, comm_ref, send_sems, recv_sems):
        my = lax.axis_index("i")
        p_a = 3 - my
        p_b = lax.bitwise_xor(my, 1)

        barrier_sem = pltpu.get_barrier_semaphore()
        for nbr in (p_a, p_b):
            pl.semaphore_signal(
                barrier_sem, inc=1,
                device_id=(nbr,), device_id_type=pl.DeviceIdType.MESH,
            )
        pl.semaphore_wait(barrier_sem, 2)

        def start(e, partner):
            rdma = pltpu.make_async_remote_copy(
                src_ref=comm_ref.at[2 * e],
                dst_ref=comm_ref.at[2 * e + 1],
                send_sem=send_sems.at[e],
                recv_sem=recv_sems.at[e],
                device_id=(partner,),
                device_id_type=pl.DeviceIdType.MESH,
            )
            rdma.start()
            return rdma

        mod = lax.dot_general(
            temb_ref[:, :], wmod_ref[:, :], (((1,), (0,)), ((), ())),
            preferred_element_type=jnp.float32,
        )

        def mod_row(b, k):
            return mod[b:b + 1, k * D:(k + 1) * D]

        wq = wq_ref[:, :]
        wk = wk_ref[:, :]
        wv = wv_ref[:, :]
        wo = wo_ref[:, :]
        x0 = [x_ref[b] for b in range(B)]

        def attn(b):
            xm = _ln_mod(x0[b], mod_row(b, 0), mod_row(b, 1))
            q = _dot(xm, wq)
            k = _dot(xm, wk)
            v = _dot(xm, wv)
            o_heads = []
            for h in range(H_PER):
                sl = slice(h * DH, (h + 1) * DH)
                s = _dot(q[:, sl], k[:, sl], trans_b=True) * 0.125
                s_max = jnp.max(s, axis=-1, keepdims=True)
                p = jnp.exp(s - s_max)
                p = p / jnp.sum(p, axis=-1, keepdims=True)
                o_heads.append(_dot(p, v[:, sl]))
            o = jnp.concatenate(o_heads, axis=1)
            return _dot(o, wo).astype(jnp.bfloat16)

        def ffn(b, x1b):
            xm2 = _ln_mod(x1b, mod_row(b, 3), mod_row(b, 4))
            h1 = _dot(xm2, wff1_ref[:, :])
            h1 = h1 * jax.nn.sigmoid(h1)
            return _dot(h1, wff2_ref[:, :]).astype(jnp.bfloat16)

        def pair_sum_bf16(e):
            return comm_ref[2 * e] + comm_ref[2 * e + 1]

        def total_f32(e):
            return (comm_ref[2 * e].astype(jnp.float32)
                    + comm_ref[2 * e + 1].astype(jnp.float32))

        comm_ref[2 * A0] = attn(0)
        xa0 = start(A0, p_a)
        comm_ref[2 * A1] = attn(1)
        xa1 = start(A1, p_a)

        xa0.wait()
        comm_ref[2 * B0] = pair_sum_bf16(A0)
        xb0 = start(B0, p_b)
        xa1.wait()
        comm_ref[2 * B1] = pair_sum_bf16(A1)
        xb1 = start(B1, p_b)

        xb0.wait()
        x1_0 = x0[0] + mod_row(0, 2) * total_f32(B0)
        comm_ref[2 * C0] = ffn(0, x1_0)
        xc0 = start(C0, p_a)
        xb1.wait()
        x1_1 = x0[1] + mod_row(1, 2) * total_f32(B1)
        comm_ref[2 * C1] = ffn(1, x1_1)
        xc1 = start(C1, p_a)

        xc0.wait()
        comm_ref[2 * D0] = pair_sum_bf16(C0)
        xd0 = start(D0, p_b)
        xc1.wait()
        comm_ref[2 * D1] = pair_sum_bf16(C1)
        xd1 = start(D1, p_b)

        xd0.wait()
        out_ref[0] = x1_0 + mod_row(0, 5) * total_f32(D0)
        xd1.wait()
        out_ref[1] = x1_1 + mod_row(1, 5) * total_f32(D1)

    return pl.pallas_call(
        body,
        out_shape=jax.ShapeDtypeStruct((B, S, D), jnp.float32),
        in_specs=[pl.BlockSpec(memory_space=pltpu.VMEM)] * 9,
        out_specs=pl.BlockSpec(memory_space=pltpu.VMEM),
        scratch_shapes=[
            pltpu.VMEM((16, S, D), jnp.bfloat16),
            pltpu.SemaphoreType.DMA((8,)),
            pltpu.SemaphoreType.DMA((8,)),
        ],
        compiler_params=pltpu.CompilerParams(collective_id=0),
    )(x, Wq, Wk, Wv, Wo, t_emb, W_mod, W_ff1, W_ff2)
